# baseline (speedup 1.0000x reference)
import functools

import jax
import jax.numpy as jnp
from jax import lax
from jax.experimental import pallas as pl
from jax.experimental.pallas import tpu as pltpu
from jax.experimental.pallas import tpu_sc as plsc

NN = 10000
NE = 320000
FD = 64
FR = 128
NEG = 0.2

NC = 2
NS = 16
NW = NC * NS
EPT = NE // NW
CH = 40
NCHUNK = EPT // CH
NP = 10240
RPT = NP // NS


def _mk_tables(h, asrce, adste):
    ge = jnp.max(asrce, axis=0, keepdims=True)
    u = ge + adste
    ube = jnp.maximum(u, NEG * u)
    t = asrce + adste
    ws = jnp.exp(jnp.maximum(t, NEG * t) - ube)
    taba = jnp.concatenate([h, asrce], axis=1)
    tabb = jnp.concatenate([adste, ube], axis=1)
    selfmsg = jnp.concatenate([h * ws, ws], axis=1)
    return taba, tabb, selfmsg


def _dense1_body(x_ref, w1_ref, a1s_ref, a1d_ref, rrep_ref,
                 taba_ref, tabb_ref, selfmsg_ref):
    x = x_ref[...]
    h = jnp.dot(x, w1_ref[...], preferred_element_type=jnp.float32)
    a_src = jnp.dot(h, a1s_ref[...], preferred_element_type=jnp.float32)
    a_dst = jnp.dot(h, a1d_ref[...], preferred_element_type=jnp.float32)
    asrce = jnp.dot(a_src, rrep_ref[...], preferred_element_type=jnp.float32)
    adste = jnp.dot(a_dst, rrep_ref[...], preferred_element_type=jnp.float32)
    taba, tabb, selfmsg = _mk_tables(h, asrce, adste)
    taba_ref[...] = taba
    tabb_ref[...] = tabb
    selfmsg_ref[...] = selfmsg


def _dense2_body(parts_ref, selfmsg_ref, m128_ref, s128_ref, b1_ref, w2_ref,
                 a2s_ref, a2d_ref,
                 taba_ref, tabb_ref, selfmsg2_ref):
    acc = parts_ref[0, :NN] + parts_ref[1, :NN] + selfmsg_ref[...]
    m1 = jnp.dot(acc, m128_ref[...], preferred_element_type=jnp.float32)
    srep = jnp.dot(acc, s128_ref[...], preferred_element_type=jnp.float32)
    h1 = jnp.maximum(m1 / (srep + 1e-16) + b1_ref[...], 0.0)
    h2 = jnp.dot(h1, w2_ref[...], preferred_element_type=jnp.float32)
    asrce = jnp.dot(h2, a2s_ref[...], preferred_element_type=jnp.float32)
    adste = jnp.dot(h2, a2d_ref[...], preferred_element_type=jnp.float32)
    taba, tabb, selfmsg = _mk_tables(h2, asrce, adste)
    taba_ref[...] = taba
    tabb_ref[...] = tabb
    selfmsg2_ref[...] = selfmsg


def _out_body(parts_ref, selfmsg_ref, m128_ref, s128_ref, b2_ref, o_ref):
    acc = parts_ref[0, :NN] + parts_ref[1, :NN] + selfmsg_ref[...]
    m2 = jnp.dot(acc, m128_ref[...], preferred_element_type=jnp.float32)
    srep = jnp.dot(acc, s128_ref[...], preferred_element_type=jnp.float32)
    o = m2 / (srep + 1e-16) + b2_ref[...]
    col = lax.broadcasted_iota(jnp.int32, (1, FD), 1)
    om = jnp.where(col < 47, o, -1e30)
    mx = jnp.max(om, axis=1, keepdims=True)
    ssum = jnp.sum(jnp.exp(om - mx), axis=1, keepdims=True)
    o_ref[...] = o - (mx + jnp.log(ssum))


def _edge_body(ei_hbm, taba_hbm, tabb_hbm, zeros_hbm, out_hbm,
               eiA, eiB, eiC, siA, siB, siC,
               sbA, dbA, mbA, sbB, dbB, mbB, sbC, dbC, mbC,
               acc, ixA, ixB, ixC, gaA, gbA, gaB, gbB, gaC, gbC,
               ssA, ssB, ssC):
    cid = lax.axis_index("c")
    sid = lax.axis_index("s")
    wid = cid * NS + sid
    pltpu.sync_copy(zeros_hbm, acc.at[pl.ds(sid * RPT, RPT)])
    ebase = wid * EPT
    plsc.subcore_barrier()

    EI = (eiA, eiB, eiC)
    SI = (siA, siB, siC)
    SB = (sbA, sbB, sbC)
    DB = (dbA, dbB, dbC)
    MB = (mbA, mbB, mbC)
    IX = (ixA, ixB, ixC)
    GA = (gaA, gaB, gaC)
    GB = (gbA, gbB, gbC)
    SS = (ssA, ssB, ssC)

    def fire_idx(c, t):
        pltpu.async_copy(
            ei_hbm.at[pl.ds(2 * (ebase + c * CH), 2 * CH)], EI[t], IX[t])

    def wait_idx(c, t):
        pltpu.make_async_copy(
            ei_hbm.at[pl.ds(2 * (ebase + c * CH), 2 * CH)], EI[t], IX[t]).wait()

    def fire_gather(c, t):
        pltpu.async_copy(taba_hbm.at[EI[t].at[pl.ds(0, CH)]], SB[t], GA[t])
        pltpu.async_copy(tabb_hbm.at[EI[t].at[pl.ds(CH, CH)]], DB[t], GB[t])

    def wait_gather(c, t):
        pltpu.make_async_copy(
            taba_hbm.at[EI[t].at[pl.ds(0, CH)]], SB[t], GA[t]).wait()
        pltpu.make_async_copy(
            tabb_hbm.at[EI[t].at[pl.ds(CH, CH)]], DB[t], GB[t]).wait()

    def copy_scat_idx(t):
        ei = EI[t]
        si = SI[t]
        si[pl.ds(0, 16)] = ei[pl.ds(CH, 16)]
        si[pl.ds(16, 16)] = ei[pl.ds(CH + 16, 16)]
        si[pl.ds(24, 16)] = ei[pl.ds(CH + 24, 16)]

    def compute(t):
        sb, db, mb = SB[t], DB[t], MB[t]

        @plsc.parallel_loop(0, CH, step=1, unroll=4)
        def edge(k):
            for s in range(4):
                o = 16 * s
                h16 = sb[k, pl.ds(o, 16)]
                se = sb[k, pl.ds(FD + o, 16)]
                de = db[k, pl.ds(o, 16)]
                ue = db[k, pl.ds(FD + o, 16)]
                t_ = se + de
                w = jnp.exp(jnp.maximum(t_, NEG * t_) - ue)
                mb[k, pl.ds(FD + o, 16)] = w
                mb[k, pl.ds(o, 16)] = h16 * w

    def fire_scatter(t):
        pltpu.async_copy(MB[t], acc.at[SI[t]], SS[t], add=True)

    def wait_scatter(t):
        pltpu.make_async_copy(MB[t], acc.at[SI[t]], SS[t]).wait()

    def slot(c, t, j):
        wait_gather(c, t)
        t1 = (t + 1) % 3
        wait_idx(c + 1, t1)
        fire_gather(c + 1, t1)

        @pl.when(j > 0)
        def _():
            wait_scatter(t)

        copy_scat_idx(t)
        compute(t)
        fire_scatter(t)

        @pl.when(c + 3 < NCHUNK)
        def _():
            fire_idx(c + 3, t)

    fire_idx(0, 0)
    fire_idx(1, 1)
    fire_idx(2, 2)
    wait_idx(0, 0)
    fire_gather(0, 0)

    def triple(j, carry):
        c = 3 * j
        slot(c, 0, j)
        slot(c + 1, 1, j)
        slot(c + 2, 2, j)
        return carry

    lax.fori_loop(0, (NCHUNK - 1) // 3, triple, 0)

    wait_gather(NCHUNK - 1, 0)
    wait_scatter(0)
    copy_scat_idx(0)
    compute(0)
    fire_scatter(0)
    wait_scatter(1)
    wait_scatter(2)
    wait_scatter(0)

    plsc.subcore_barrier()
    pltpu.sync_copy(acc.at[pl.ds(sid * RPT, RPT)],
                    out_hbm.at[cid, pl.ds(sid * RPT, RPT)])


_edge_kernel = functools.partial(
    pl.kernel,
    out_type=jax.ShapeDtypeStruct((NC, NP, FR), jnp.float32),
    mesh=plsc.VectorSubcoreMesh(core_axis_name="c", subcore_axis_name="s"),
    compiler_params=pltpu.CompilerParams(needs_layout_passes=False),
    scratch_types=[
        pltpu.VMEM((2 * CH,), jnp.int32),
        pltpu.VMEM((2 * CH,), jnp.int32),
        pltpu.VMEM((2 * CH,), jnp.int32),
        pltpu.VMEM((CH,), jnp.int32),
        pltpu.VMEM((CH,), jnp.int32),
        pltpu.VMEM((CH,), jnp.int32),
        pltpu.VMEM((CH, FR), jnp.float32),
        pltpu.VMEM((CH, FR), jnp.float32),
        pltpu.VMEM((CH, FR), jnp.float32),
        pltpu.VMEM((CH, FR), jnp.float32),
        pltpu.VMEM((CH, FR), jnp.float32),
        pltpu.VMEM((CH, FR), jnp.float32),
        pltpu.VMEM((CH, FR), jnp.float32),
        pltpu.VMEM((CH, FR), jnp.float32),
        pltpu.VMEM((CH, FR), jnp.float32),
        pltpu.VMEM_SHARED((NP, FR), jnp.float32),
        pltpu.SemaphoreType.DMA,
        pltpu.SemaphoreType.DMA,
        pltpu.SemaphoreType.DMA,
        pltpu.SemaphoreType.DMA,
        pltpu.SemaphoreType.DMA,
        pltpu.SemaphoreType.DMA,
        pltpu.SemaphoreType.DMA,
        pltpu.SemaphoreType.DMA,
        pltpu.SemaphoreType.DMA,
        pltpu.SemaphoreType.DMA,
        pltpu.SemaphoreType.DMA,
        pltpu.SemaphoreType.DMA,
    ],
)(_edge_body)


def kernel(x, edge_index, W1, att_src1, att_dst1, b1, W2, att_src2, att_dst2, b2):
    f32 = jnp.float32
    ei_packed = jnp.swapaxes(edge_index.reshape(2, NE // CH, CH),
                             0, 1).reshape(2 * NE)
    eye8 = jnp.eye(8, dtype=f32)
    a1s = (eye8[:, None, :] * att_src1[:, :, None]).reshape(FD, 8)
    a1d = (eye8[:, None, :] * att_dst1[:, :, None]).reshape(FD, 8)
    rrep = jnp.kron(eye8, jnp.ones((1, 8), f32))
    m128 = jnp.concatenate([jnp.eye(FD, dtype=f32),
                            jnp.zeros((FD, FD), f32)], axis=0)
    s128 = jnp.concatenate([jnp.zeros((FD, FD), f32),
                            jnp.eye(FD, dtype=f32)], axis=0)
    w2p = jnp.zeros((FD, FD), f32).at[:, :47].set(W2)
    a2s = jnp.zeros((FD,), f32).at[:47].set(att_src2[0])
    a2d = jnp.zeros((FD,), f32).at[:47].set(att_dst2[0])
    a2se = jnp.broadcast_to(a2s[:, None], (FD, FD))
    a2de = jnp.broadcast_to(a2d[:, None], (FD, FD))
    b1r = b1.reshape(1, FD)
    b2p = jnp.zeros((1, FD), f32).at[0, :47].set(b2)
    zeros_blk = jnp.zeros((RPT, FR), f32)

    taba1, tabb1, selfmsg1 = pl.pallas_call(
        _dense1_body,
        out_shape=[
            jax.ShapeDtypeStruct((NN, FR), f32),
            jax.ShapeDtypeStruct((NN, FR), f32),
            jax.ShapeDtypeStruct((NN, FR), f32),
        ],
    )(x, W1, a1s, a1d, rrep)

    parts1 = _edge_kernel(ei_packed, taba1, tabb1, zeros_blk)

    taba2, tabb2, selfmsg2 = pl.pallas_call(
        _dense2_body,
        out_shape=[
            jax.ShapeDtypeStruct((NN, FR), f32),
            jax.ShapeDtypeStruct((NN, FR), f32),
            jax.ShapeDtypeStruct((NN, FR), f32),
        ],
    )(parts1, selfmsg1, m128, s128, b1r, w2p, a2se, a2de)

    parts2 = _edge_kernel(ei_packed, taba2, tabb2, zeros_blk)

    out = pl.pallas_call(
        _out_body,
        out_shape=jax.ShapeDtypeStruct((NN, FD), f32),
    )(parts2, selfmsg2, m128, s128, b2p)
    return out[:, :47]


# speedup vs baseline: 1.0821x; 1.0821x over previous
import functools

import jax
import jax.numpy as jnp
from jax import lax
from jax.experimental import pallas as pl
from jax.experimental.pallas import tpu as pltpu
from jax.experimental.pallas import tpu_sc as plsc

NN = 10000
NE = 320000
FD = 64
FR = 128
NEG = 0.2

NC = 2
NS = 16
NW = NC * NS
EPT = NE // NW
CH = 40
NCHUNK = EPT // CH
NP = 10240
RPT = NP // NS


def _mk_tables(h, asrce, adste):
    ge = jnp.max(asrce, axis=0, keepdims=True)
    u = ge + adste
    ube = jnp.maximum(u, NEG * u)
    t = asrce + adste
    ws = jnp.exp(jnp.maximum(t, NEG * t) - ube)
    taba = jnp.concatenate([h, asrce], axis=1)
    tabb = jnp.concatenate([adste, ube], axis=1)
    selfmsg = jnp.concatenate([h * ws, ws], axis=1)
    return taba, tabb, selfmsg


def _dense1_body(x_ref, w1_ref, a1s_ref, a1d_ref, rrep_ref,
                 taba_ref, tabb_ref, selfmsg_ref):
    x = x_ref[...]
    h = jnp.dot(x, w1_ref[...], preferred_element_type=jnp.float32)
    a_src = jnp.dot(h, a1s_ref[...], preferred_element_type=jnp.float32)
    a_dst = jnp.dot(h, a1d_ref[...], preferred_element_type=jnp.float32)
    asrce = jnp.dot(a_src, rrep_ref[...], preferred_element_type=jnp.float32)
    adste = jnp.dot(a_dst, rrep_ref[...], preferred_element_type=jnp.float32)
    taba, tabb, selfmsg = _mk_tables(h, asrce, adste)
    taba_ref[...] = taba
    tabb_ref[...] = tabb
    selfmsg_ref[...] = selfmsg


def _dense2_body(parts_ref, selfmsg_ref, m128_ref, s128_ref, b1_ref, w2_ref,
                 a2s_ref, a2d_ref,
                 taba_ref, tabb_ref, selfmsg2_ref):
    acc = parts_ref[0, :NN] + parts_ref[1, :NN] + selfmsg_ref[...]
    m1 = jnp.dot(acc, m128_ref[...], preferred_element_type=jnp.float32)
    srep = jnp.dot(acc, s128_ref[...], preferred_element_type=jnp.float32)
    h1 = jnp.maximum(m1 / (srep + 1e-16) + b1_ref[...], 0.0)
    h2 = jnp.dot(h1, w2_ref[...], preferred_element_type=jnp.float32)
    asrce = jnp.dot(h2, a2s_ref[...], preferred_element_type=jnp.float32)
    adste = jnp.dot(h2, a2d_ref[...], preferred_element_type=jnp.float32)
    taba, tabb, selfmsg = _mk_tables(h2, asrce, adste)
    taba_ref[...] = taba
    tabb_ref[...] = tabb
    selfmsg2_ref[...] = selfmsg


def _out_body(parts_ref, selfmsg_ref, m128_ref, s128_ref, b2_ref, o_ref):
    acc = parts_ref[0, :NN] + parts_ref[1, :NN] + selfmsg_ref[...]
    m2 = jnp.dot(acc, m128_ref[...], preferred_element_type=jnp.float32)
    srep = jnp.dot(acc, s128_ref[...], preferred_element_type=jnp.float32)
    o = m2 / (srep + 1e-16) + b2_ref[...]
    col = lax.broadcasted_iota(jnp.int32, (1, FD), 1)
    om = jnp.where(col < 47, o, -1e30)
    mx = jnp.max(om, axis=1, keepdims=True)
    ssum = jnp.sum(jnp.exp(om - mx), axis=1, keepdims=True)
    o_ref[...] = o - (mx + jnp.log(ssum))


def _edge_body(ei_hbm, taba_hbm, tabb_hbm, zeros_hbm, out_hbm,
               eiA, eiB, eiC, siA, siB, siC,
               sbA, dbA, mbA, sbB, dbB, mbB, sbC, dbC, mbC,
               acc, ixA, ixB, ixC, gaA, gbA, gaB, gbB, gaC, gbC,
               ssA, ssB, ssC):
    cid = lax.axis_index("c")
    sid = lax.axis_index("s")
    wid = cid * NS + sid
    pltpu.sync_copy(zeros_hbm, acc.at[pl.ds(sid * RPT, RPT)])
    ebase = wid * EPT
    plsc.subcore_barrier()

    EI = (eiA, eiB, eiC)
    SI = (siA, siB, siC)
    SB = (sbA, sbB, sbC)
    DB = (dbA, dbB, dbC)
    MB = (mbA, mbB, mbC)
    IX = (ixA, ixB, ixC)
    GA = (gaA, gaB, gaC)
    GB = (gbA, gbB, gbC)
    SS = (ssA, ssB, ssC)

    def fire_idx(c, t):
        pltpu.async_copy(
            ei_hbm.at[pl.ds(2 * (ebase + c * CH), 2 * CH)], EI[t], IX[t])

    def wait_idx(c, t):
        pltpu.make_async_copy(
            ei_hbm.at[pl.ds(2 * (ebase + c * CH), 2 * CH)], EI[t], IX[t]).wait()

    def fire_gather(c, t):
        pltpu.async_copy(taba_hbm.at[EI[t].at[pl.ds(0, CH)]], SB[t], GA[t])
        pltpu.async_copy(tabb_hbm.at[EI[t].at[pl.ds(CH, CH)]], DB[t], GB[t])

    def wait_gather(c, t):
        pltpu.make_async_copy(
            taba_hbm.at[EI[t].at[pl.ds(0, CH)]], SB[t], GA[t]).wait()
        pltpu.make_async_copy(
            tabb_hbm.at[EI[t].at[pl.ds(CH, CH)]], DB[t], GB[t]).wait()

    def copy_scat_idx(t):
        ei = EI[t]
        si = SI[t]
        si[pl.ds(0, 16)] = ei[pl.ds(CH, 16)]
        si[pl.ds(16, 16)] = ei[pl.ds(CH + 16, 16)]
        si[pl.ds(24, 16)] = ei[pl.ds(CH + 24, 16)]

    def compute(t):
        sb, db, mb = SB[t], DB[t], MB[t]

        @plsc.parallel_loop(0, CH, step=1, unroll=4)
        def edge(k):
            for s in range(4):
                o = 16 * s
                h16 = sb[k, pl.ds(o, 16)]
                se = sb[k, pl.ds(FD + o, 16)]
                de = db[k, pl.ds(o, 16)]
                ue = db[k, pl.ds(FD + o, 16)]
                t_ = se + de
                w = jnp.exp(jnp.maximum(t_, NEG * t_) - ue)
                mb[k, pl.ds(FD + o, 16)] = w
                mb[k, pl.ds(o, 16)] = h16 * w

    def fire_scatter(t):
        pltpu.async_copy(MB[t], acc.at[SI[t]], SS[t], add=True)

    def wait_scatter(t):
        pltpu.make_async_copy(MB[t], acc.at[SI[t]], SS[t]).wait()

    def slot(c, t, j):
        wait_gather(c, t)
        t2 = (t + 2) % 3

        @pl.when(c + 2 < NCHUNK)
        def _():
            wait_idx(c + 2, t2)
            fire_gather(c + 2, t2)

        @pl.when(j > 0)
        def _():
            wait_scatter(t)

        copy_scat_idx(t)
        compute(t)
        fire_scatter(t)

        @pl.when(c + 3 < NCHUNK)
        def _():
            fire_idx(c + 3, t)

    fire_idx(0, 0)
    fire_idx(1, 1)
    fire_idx(2, 2)
    wait_idx(0, 0)
    fire_gather(0, 0)
    wait_idx(1, 1)
    fire_gather(1, 1)

    def triple(j, carry):
        c = 3 * j
        slot(c, 0, j)
        slot(c + 1, 1, j)
        slot(c + 2, 2, j)
        return carry

    lax.fori_loop(0, (NCHUNK - 1) // 3, triple, 0)

    wait_gather(NCHUNK - 1, 0)
    wait_scatter(0)
    copy_scat_idx(0)
    compute(0)
    fire_scatter(0)
    wait_scatter(1)
    wait_scatter(2)
    wait_scatter(0)

    plsc.subcore_barrier()
    pltpu.sync_copy(acc.at[pl.ds(sid * RPT, RPT)],
                    out_hbm.at[cid, pl.ds(sid * RPT, RPT)])


_edge_kernel = functools.partial(
    pl.kernel,
    out_type=jax.ShapeDtypeStruct((NC, NP, FR), jnp.float32),
    mesh=plsc.VectorSubcoreMesh(core_axis_name="c", subcore_axis_name="s"),
    compiler_params=pltpu.CompilerParams(needs_layout_passes=False),
    scratch_types=[
        pltpu.VMEM((2 * CH,), jnp.int32),
        pltpu.VMEM((2 * CH,), jnp.int32),
        pltpu.VMEM((2 * CH,), jnp.int32),
        pltpu.VMEM((CH,), jnp.int32),
        pltpu.VMEM((CH,), jnp.int32),
        pltpu.VMEM((CH,), jnp.int32),
        pltpu.VMEM((CH, FR), jnp.float32),
        pltpu.VMEM((CH, FR), jnp.float32),
        pltpu.VMEM((CH, FR), jnp.float32),
        pltpu.VMEM((CH, FR), jnp.float32),
        pltpu.VMEM((CH, FR), jnp.float32),
        pltpu.VMEM((CH, FR), jnp.float32),
        pltpu.VMEM((CH, FR), jnp.float32),
        pltpu.VMEM((CH, FR), jnp.float32),
        pltpu.VMEM((CH, FR), jnp.float32),
        pltpu.VMEM_SHARED((NP, FR), jnp.float32),
        pltpu.SemaphoreType.DMA,
        pltpu.SemaphoreType.DMA,
        pltpu.SemaphoreType.DMA,
        pltpu.SemaphoreType.DMA,
        pltpu.SemaphoreType.DMA,
        pltpu.SemaphoreType.DMA,
        pltpu.SemaphoreType.DMA,
        pltpu.SemaphoreType.DMA,
        pltpu.SemaphoreType.DMA,
        pltpu.SemaphoreType.DMA,
        pltpu.SemaphoreType.DMA,
        pltpu.SemaphoreType.DMA,
    ],
)(_edge_body)


def kernel(x, edge_index, W1, att_src1, att_dst1, b1, W2, att_src2, att_dst2, b2):
    f32 = jnp.float32
    ei_packed = jnp.swapaxes(edge_index.reshape(2, NE // CH, CH),
                             0, 1).reshape(2 * NE)
    eye8 = jnp.eye(8, dtype=f32)
    a1s = (eye8[:, None, :] * att_src1[:, :, None]).reshape(FD, 8)
    a1d = (eye8[:, None, :] * att_dst1[:, :, None]).reshape(FD, 8)
    rrep = jnp.kron(eye8, jnp.ones((1, 8), f32))
    m128 = jnp.concatenate([jnp.eye(FD, dtype=f32),
                            jnp.zeros((FD, FD), f32)], axis=0)
    s128 = jnp.concatenate([jnp.zeros((FD, FD), f32),
                            jnp.eye(FD, dtype=f32)], axis=0)
    w2p = jnp.zeros((FD, FD), f32).at[:, :47].set(W2)
    a2s = jnp.zeros((FD,), f32).at[:47].set(att_src2[0])
    a2d = jnp.zeros((FD,), f32).at[:47].set(att_dst2[0])
    a2se = jnp.broadcast_to(a2s[:, None], (FD, FD))
    a2de = jnp.broadcast_to(a2d[:, None], (FD, FD))
    b1r = b1.reshape(1, FD)
    b2p = jnp.zeros((1, FD), f32).at[0, :47].set(b2)
    zeros_blk = jnp.zeros((RPT, FR), f32)

    taba1, tabb1, selfmsg1 = pl.pallas_call(
        _dense1_body,
        out_shape=[
            jax.ShapeDtypeStruct((NN, FR), f32),
            jax.ShapeDtypeStruct((NN, FR), f32),
            jax.ShapeDtypeStruct((NN, FR), f32),
        ],
    )(x, W1, a1s, a1d, rrep)

    parts1 = _edge_kernel(ei_packed, taba1, tabb1, zeros_blk)

    taba2, tabb2, selfmsg2 = pl.pallas_call(
        _dense2_body,
        out_shape=[
            jax.ShapeDtypeStruct((NN, FR), f32),
            jax.ShapeDtypeStruct((NN, FR), f32),
            jax.ShapeDtypeStruct((NN, FR), f32),
        ],
    )(parts1, selfmsg1, m128, s128, b1r, w2p, a2se, a2de)

    parts2 = _edge_kernel(ei_packed, taba2, tabb2, zeros_blk)

    out = pl.pallas_call(
        _out_body,
        out_shape=jax.ShapeDtypeStruct((NN, FD), f32),
    )(parts2, selfmsg2, m128, s128, b2p)
    return out[:, :47]


# speedup vs baseline: 1.0856x; 1.0033x over previous
import functools

import jax
import jax.numpy as jnp
from jax import lax
from jax.experimental import pallas as pl
from jax.experimental.pallas import tpu as pltpu
from jax.experimental.pallas import tpu_sc as plsc

NN = 10000
NE = 320000
FD = 64
FR = 128
NEG = 0.2

NC = 2
NS = 16
NW = NC * NS
EPT = NE // NW
CH = 40
NCHUNK = EPT // CH
NP = 10240
RPT = NP // NS


def _mk_tables(h, asrce, adste):
    ge = jnp.max(asrce, axis=0, keepdims=True)
    u = ge + adste
    ube = jnp.maximum(u, NEG * u)
    t = asrce + adste
    ws = jnp.exp(jnp.maximum(t, NEG * t) - ube)
    taba = jnp.concatenate([h, asrce], axis=1)
    tabb = jnp.concatenate([adste, ube], axis=1)
    selfmsg = jnp.concatenate([h * ws, ws], axis=1)
    return taba, tabb, selfmsg


def _dense1_body(x_ref, w1_ref, a1s_ref, a1d_ref, rrep_ref,
                 taba_ref, tabb_ref, selfmsg_ref):
    x = x_ref[...]
    h = jnp.dot(x, w1_ref[...], preferred_element_type=jnp.float32)
    a_src = jnp.dot(h, a1s_ref[...], preferred_element_type=jnp.float32)
    a_dst = jnp.dot(h, a1d_ref[...], preferred_element_type=jnp.float32)
    asrce = jnp.dot(a_src, rrep_ref[...], preferred_element_type=jnp.float32)
    adste = jnp.dot(a_dst, rrep_ref[...], preferred_element_type=jnp.float32)
    taba, tabb, selfmsg = _mk_tables(h, asrce, adste)
    taba_ref[...] = taba
    tabb_ref[...] = tabb
    selfmsg_ref[...] = selfmsg


def _dense2_body(parts_ref, selfmsg_ref, m128_ref, s128_ref, b1_ref, w2_ref,
                 a2s_ref, a2d_ref,
                 taba_ref, tabb_ref, selfmsg2_ref):
    acc = parts_ref[0, :NN] + parts_ref[1, :NN] + selfmsg_ref[...]
    m1 = jnp.dot(acc, m128_ref[...], preferred_element_type=jnp.float32)
    srep = jnp.dot(acc, s128_ref[...], preferred_element_type=jnp.float32)
    h1 = jnp.maximum(m1 / (srep + 1e-16) + b1_ref[...], 0.0)
    h2 = jnp.dot(h1, w2_ref[...], preferred_element_type=jnp.float32)
    asrce = jnp.dot(h2, a2s_ref[...], preferred_element_type=jnp.float32)
    adste = jnp.dot(h2, a2d_ref[...], preferred_element_type=jnp.float32)
    taba, tabb, selfmsg = _mk_tables(h2, asrce, adste)
    taba_ref[...] = taba
    tabb_ref[...] = tabb
    selfmsg2_ref[...] = selfmsg


def _out_body(parts_ref, selfmsg_ref, m128_ref, s128_ref, b2_ref, o_ref):
    acc = parts_ref[0, :NN] + parts_ref[1, :NN] + selfmsg_ref[...]
    m2 = jnp.dot(acc, m128_ref[...], preferred_element_type=jnp.float32)
    srep = jnp.dot(acc, s128_ref[...], preferred_element_type=jnp.float32)
    o = m2 / (srep + 1e-16) + b2_ref[...]
    col = lax.broadcasted_iota(jnp.int32, (1, FD), 1)
    om = jnp.where(col < 47, o, -1e30)
    mx = jnp.max(om, axis=1, keepdims=True)
    ssum = jnp.sum(jnp.exp(om - mx), axis=1, keepdims=True)
    o_ref[...] = o - (mx + jnp.log(ssum))


def _edge_body(ei_hbm, taba_hbm, tabb_hbm, zeros_hbm, out_hbm,
               eiA, eiB, eiC, siA, siB, siC,
               sbA, dbA, mbA, sbB, dbB, mbB, sbC, dbC, mbC,
               acc, ixA, ixB, ixC, gaA, gbA, gaB, gbB, gaC, gbC,
               ssA, ssB, ssC):
    cid = lax.axis_index("c")
    sid = lax.axis_index("s")
    wid = cid * NS + sid
    pltpu.sync_copy(zeros_hbm, acc.at[pl.ds(sid * RPT, RPT)])
    ebase = wid * EPT
    plsc.subcore_barrier()

    EI = (eiA, eiB, eiC)
    SI = (siA, siB, siC)
    SB = (sbA, sbB, sbC)
    DB = (dbA, dbB, dbC)
    MB = (mbA, mbB, mbC)
    IX = (ixA, ixB, ixC)
    GA = (gaA, gaB, gaC)
    GB = (gbA, gbB, gbC)
    SS = (ssA, ssB, ssC)

    def fire_idx(c, t):
        pltpu.async_copy(
            ei_hbm.at[pl.ds(2 * (ebase + c * CH), 2 * CH)], EI[t], IX[t])

    def wait_idx(c, t):
        pltpu.make_async_copy(
            ei_hbm.at[pl.ds(2 * (ebase + c * CH), 2 * CH)], EI[t], IX[t]).wait()

    def fire_gather(c, t):
        pltpu.async_copy(taba_hbm.at[EI[t].at[pl.ds(0, CH)]], SB[t], GA[t])
        pltpu.async_copy(tabb_hbm.at[EI[t].at[pl.ds(CH, CH)]], DB[t], GB[t])

    def wait_gather(c, t):
        pltpu.make_async_copy(
            taba_hbm.at[EI[t].at[pl.ds(0, CH)]], SB[t], GA[t]).wait()
        pltpu.make_async_copy(
            tabb_hbm.at[EI[t].at[pl.ds(CH, CH)]], DB[t], GB[t]).wait()

    def copy_scat_idx(t):
        ei = EI[t]
        si = SI[t]
        si[pl.ds(0, 16)] = ei[pl.ds(CH, 16)]
        si[pl.ds(16, 16)] = ei[pl.ds(CH + 16, 16)]
        si[pl.ds(24, 16)] = ei[pl.ds(CH + 24, 16)]

    def compute(t):
        sb, db, mb = SB[t], DB[t], MB[t]

        @plsc.parallel_loop(0, CH, step=1, unroll=8)
        def edge(k):
            for s in range(4):
                o = 16 * s
                h16 = sb[k, pl.ds(o, 16)]
                se = sb[k, pl.ds(FD + o, 16)]
                de = db[k, pl.ds(o, 16)]
                ue = db[k, pl.ds(FD + o, 16)]
                t_ = se + de
                w = jnp.exp(jnp.maximum(t_, NEG * t_) - ue)
                mb[k, pl.ds(FD + o, 16)] = w
                mb[k, pl.ds(o, 16)] = h16 * w

    def fire_scatter(t):
        pltpu.async_copy(MB[t], acc.at[SI[t]], SS[t], add=True)

    def wait_scatter(t):
        pltpu.make_async_copy(MB[t], acc.at[SI[t]], SS[t]).wait()

    def slot(c, t, j):
        wait_gather(c, t)
        t2 = (t + 2) % 3

        @pl.when(c + 2 < NCHUNK)
        def _():
            wait_idx(c + 2, t2)
            fire_gather(c + 2, t2)

        @pl.when(j > 0)
        def _():
            wait_scatter(t)

        copy_scat_idx(t)
        compute(t)
        fire_scatter(t)

        @pl.when(c + 3 < NCHUNK)
        def _():
            fire_idx(c + 3, t)

    fire_idx(0, 0)
    fire_idx(1, 1)
    fire_idx(2, 2)
    wait_idx(0, 0)
    fire_gather(0, 0)
    wait_idx(1, 1)
    fire_gather(1, 1)

    def triple(j, carry):
        c = 3 * j
        slot(c, 0, j)
        slot(c + 1, 1, j)
        slot(c + 2, 2, j)
        return carry

    lax.fori_loop(0, (NCHUNK - 1) // 3, triple, 0)

    wait_gather(NCHUNK - 1, 0)
    wait_scatter(0)
    copy_scat_idx(0)
    compute(0)
    fire_scatter(0)
    wait_scatter(1)
    wait_scatter(2)
    wait_scatter(0)

    plsc.subcore_barrier()
    pltpu.sync_copy(acc.at[pl.ds(sid * RPT, RPT)],
                    out_hbm.at[cid, pl.ds(sid * RPT, RPT)])


_edge_kernel = functools.partial(
    pl.kernel,
    out_type=jax.ShapeDtypeStruct((NC, NP, FR), jnp.float32),
    mesh=plsc.VectorSubcoreMesh(core_axis_name="c", subcore_axis_name="s"),
    compiler_params=pltpu.CompilerParams(needs_layout_passes=False),
    scratch_types=[
        pltpu.VMEM((2 * CH,), jnp.int32),
        pltpu.VMEM((2 * CH,), jnp.int32),
        pltpu.VMEM((2 * CH,), jnp.int32),
        pltpu.VMEM((CH,), jnp.int32),
        pltpu.VMEM((CH,), jnp.int32),
        pltpu.VMEM((CH,), jnp.int32),
        pltpu.VMEM((CH, FR), jnp.float32),
        pltpu.VMEM((CH, FR), jnp.float32),
        pltpu.VMEM((CH, FR), jnp.float32),
        pltpu.VMEM((CH, FR), jnp.float32),
        pltpu.VMEM((CH, FR), jnp.float32),
        pltpu.VMEM((CH, FR), jnp.float32),
        pltpu.VMEM((CH, FR), jnp.float32),
        pltpu.VMEM((CH, FR), jnp.float32),
        pltpu.VMEM((CH, FR), jnp.float32),
        pltpu.VMEM_SHARED((NP, FR), jnp.float32),
        pltpu.SemaphoreType.DMA,
        pltpu.SemaphoreType.DMA,
        pltpu.SemaphoreType.DMA,
        pltpu.SemaphoreType.DMA,
        pltpu.SemaphoreType.DMA,
        pltpu.SemaphoreType.DMA,
        pltpu.SemaphoreType.DMA,
        pltpu.SemaphoreType.DMA,
        pltpu.SemaphoreType.DMA,
        pltpu.SemaphoreType.DMA,
        pltpu.SemaphoreType.DMA,
        pltpu.SemaphoreType.DMA,
    ],
)(_edge_body)


def kernel(x, edge_index, W1, att_src1, att_dst1, b1, W2, att_src2, att_dst2, b2):
    f32 = jnp.float32
    ei_packed = jnp.swapaxes(edge_index.reshape(2, NE // CH, CH),
                             0, 1).reshape(2 * NE)
    eye8 = jnp.eye(8, dtype=f32)
    a1s = (eye8[:, None, :] * att_src1[:, :, None]).reshape(FD, 8)
    a1d = (eye8[:, None, :] * att_dst1[:, :, None]).reshape(FD, 8)
    rrep = jnp.kron(eye8, jnp.ones((1, 8), f32))
    m128 = jnp.concatenate([jnp.eye(FD, dtype=f32),
                            jnp.zeros((FD, FD), f32)], axis=0)
    s128 = jnp.concatenate([jnp.zeros((FD, FD), f32),
                            jnp.eye(FD, dtype=f32)], axis=0)
    w2p = jnp.zeros((FD, FD), f32).at[:, :47].set(W2)
    a2s = jnp.zeros((FD,), f32).at[:47].set(att_src2[0])
    a2d = jnp.zeros((FD,), f32).at[:47].set(att_dst2[0])
    a2se = jnp.broadcast_to(a2s[:, None], (FD, FD))
    a2de = jnp.broadcast_to(a2d[:, None], (FD, FD))
    b1r = b1.reshape(1, FD)
    b2p = jnp.zeros((1, FD), f32).at[0, :47].set(b2)
    zeros_blk = jnp.zeros((RPT, FR), f32)

    taba1, tabb1, selfmsg1 = pl.pallas_call(
        _dense1_body,
        out_shape=[
            jax.ShapeDtypeStruct((NN, FR), f32),
            jax.ShapeDtypeStruct((NN, FR), f32),
            jax.ShapeDtypeStruct((NN, FR), f32),
        ],
    )(x, W1, a1s, a1d, rrep)

    parts1 = _edge_kernel(ei_packed, taba1, tabb1, zeros_blk)

    taba2, tabb2, selfmsg2 = pl.pallas_call(
        _dense2_body,
        out_shape=[
            jax.ShapeDtypeStruct((NN, FR), f32),
            jax.ShapeDtypeStruct((NN, FR), f32),
            jax.ShapeDtypeStruct((NN, FR), f32),
        ],
    )(parts1, selfmsg1, m128, s128, b1r, w2p, a2se, a2de)

    parts2 = _edge_kernel(ei_packed, taba2, tabb2, zeros_blk)

    out = pl.pallas_call(
        _out_body,
        out_shape=jax.ShapeDtypeStruct((NN, FD), f32),
    )(parts2, selfmsg2, m128, s128, b2p)
    return out[:, :47]
